# baseline (speedup 1.0000x reference)
import functools

import jax
import jax.numpy as jnp
from jax import lax
from jax.experimental import pallas as pl
from jax.experimental.pallas import tpu as pltpu
from jax.experimental.pallas import tpu_sc as plsc

K = 1024
D = 32
H = 64
COMMIT = 0.25
EPS = 1e-5

NC, NS, L = 2, 16, 16
NW = NC * NS
M = 8 * 8192
BPW = M // NW
CHUNK = 128
NCHUNK = BPW // CHUNK


def _tables_body(W1, b1, ln_g, ln_b, W2, b2, cb, qst_ref, idx_ref, err_ref):
    vals = lax.broadcasted_iota(jnp.int32, (K, 1), 0).astype(jnp.float32)
    norm = vals / (K - 1) * 2.0 - 1.0
    h = norm * W1[...] + b1[...]
    mu = jnp.mean(h, axis=1, keepdims=True)
    var = jnp.mean((h - mu) ** 2, axis=1, keepdims=True)
    h = (h - mu) / jnp.sqrt(var + EPS) * ln_g[...] + ln_b[...]
    h = jnp.maximum(h, 0.0)
    z = jnp.dot(h, W2[...], preferred_element_type=jnp.float32) + b2[...]

    c = cb[...]
    zc = lax.dot_general(z, c, (((1,), (1,)), ((), ())),
                         preferred_element_type=jnp.float32)
    dist = (jnp.sum(z * z, axis=1, keepdims=True)
            - 2.0 * zc
            + jnp.sum(c * c, axis=1)[None, :])
    col = lax.broadcasted_iota(jnp.int32, (K, K), 1)
    idx = jnp.argmin(dist, axis=1).astype(jnp.int32)
    onehot = (idx[:, None] == col).astype(jnp.float32)
    q = jnp.dot(onehot, c, preferred_element_type=jnp.float32)
    qst = z + (q - z)
    err = jnp.sum((q - z) ** 2, axis=1)

    qst_ref[...] = qst
    idx_ref[...] = idx.reshape(8, K // 8)
    err_ref[...] = err.reshape(8, K // 8)


def _build_tables(W1, b1, ln_g, ln_b, W2, b2, cb):
    qst, idx8, err8 = pl.pallas_call(
        _tables_body,
        out_shape=(
            jax.ShapeDtypeStruct((K, D), jnp.float32),
            jax.ShapeDtypeStruct((8, K // 8), jnp.int32),
            jax.ShapeDtypeStruct((8, K // 8), jnp.float32),
        ),
    )(W1, b1, ln_g, ln_b, W2, b2, cb)
    return qst, idx8.reshape(K), err8.reshape(K)


def _sc_gather(t1d, qst, idx_tab, err_tab, nb):
    mesh = plsc.VectorSubcoreMesh(core_axis_name="c", subcore_axis_name="s")
    mh = nb * 8192
    bpw = mh // NW
    nchunk = bpw // CHUNK

    @functools.partial(
        pl.kernel,
        mesh=mesh,
        compiler_params=pltpu.CompilerParams(use_tc_tiling_on_sc=False),
        out_type=(
            jax.ShapeDtypeStruct((nb, 8192, D), jnp.float32),
            jax.ShapeDtypeStruct((mh,), jnp.int32),
            jax.ShapeDtypeStruct((NW * L,), jnp.float32),
        ),
        scratch_types=[
            pltpu.VMEM((bpw,), jnp.int32),
            pltpu.VMEM_SHARED((K, D), jnp.float32),
            pltpu.VMEM_SHARED((K,), jnp.int32),
            pltpu.VMEM_SHARED((K,), jnp.float32),
            pltpu.VMEM((bpw, D), jnp.float32),
            pltpu.VMEM((bpw,), jnp.int32),
            pltpu.VMEM((bpw,), jnp.float32),
            pltpu.VMEM((L,), jnp.float32),
            pltpu.SemaphoreType.DMA,
            pltpu.SemaphoreType.DMA,
        ],
    )
    def k(t_hbm, qst_hbm, idxtab_hbm, errtab_hbm,
          q_out, idx_out, part_out,
          t_v, qst_v, itab_v, etab_v, rows_v, oidx_v, oerr_v, acc_v,
          sem_a, sem_b):
        sid = lax.axis_index("s")
        wid = sid * NC + lax.axis_index("c")
        BPW, NCHUNK = bpw, nchunk
        b = wid // (NW // nb)
        off = (wid % (NW // nb)) * BPW

        t_copy = pltpu.make_async_copy(
            t_hbm.at[pl.ds(wid * BPW, BPW)], t_v, sem_a)
        t_copy.start()

        @pl.when(sid == 0)
        def _():
            stage = [
                pltpu.make_async_copy(qst_hbm, qst_v, sem_b),
                pltpu.make_async_copy(idxtab_hbm, itab_v, sem_b),
                pltpu.make_async_copy(errtab_hbm, etab_v, sem_b),
            ]
            for c in stage:
                c.start()
            for c in stage:
                c.wait()

        t_copy.wait()
        plsc.subcore_barrier()

        small = []
        big = []
        for j in range(NCHUNK):
            idx_ref = t_v.at[pl.ds(j * CHUNK, CHUNK)]
            sl = pl.ds(j * CHUNK, CHUNK)
            small.append(pltpu.make_async_copy(
                itab_v.at[idx_ref], oidx_v.at[sl], sem_b))
            small.append(pltpu.make_async_copy(
                etab_v.at[idx_ref], oerr_v.at[sl], sem_b))
            big.append(pltpu.make_async_copy(
                qst_v.at[idx_ref], rows_v.at[sl], sem_a))
        for c in small:
            c.start()
        for c in big:
            c.start()
        for c in small:
            c.wait()

        idx_out_copy = pltpu.make_async_copy(
            oidx_v, idx_out.at[pl.ds(wid * BPW, BPW)], sem_b)
        idx_out_copy.start()

        def body(i, acc):
            return acc + oerr_v[pl.ds(i * L, L)]

        acc = lax.fori_loop(0, BPW // L, body, jnp.zeros((L,), jnp.float32))
        acc_v[...] = acc
        pltpu.sync_copy(acc_v, part_out.at[pl.ds(wid * L, L)])

        for c in big:
            c.wait()
        idx_out_copy.wait()
        pltpu.sync_copy(rows_v, q_out.at[b, pl.ds(off, BPW)])

    return k(t1d, qst, idx_tab, err_tab)


def _loss_body(part_ref, out_ref):
    s = jnp.sum(part_ref[...], keepdims=True)
    out_ref[...] = (1.0 + COMMIT) * s.reshape(1, 1) / jnp.float32(M * D)


def _finalize_loss(partials):
    out = pl.pallas_call(
        _loss_body,
        out_shape=jax.ShapeDtypeStruct((1, 1), jnp.float32),
    )(partials)
    return out.reshape(())


def kernel(t, W1, b1, ln_g, ln_b, W2, b2, codebook):
    B, N = t.shape[0], t.shape[1]
    qst, idx_tab, err_tab = _build_tables(
        W1, b1.reshape(1, H), ln_g.reshape(1, H), ln_b.reshape(1, H),
        W2, b2.reshape(1, D), codebook)
    q3d, idx_flat, partials = _sc_gather(
        t.reshape(M), qst, idx_tab, err_tab, B)
    loss = _finalize_loss(partials.reshape(NW, L))
    return (q3d, idx_flat.reshape(B, N), loss)


# speedup vs baseline: 1.0194x; 1.0194x over previous
import functools

import jax
import jax.numpy as jnp
from jax import lax
from jax.experimental import pallas as pl
from jax.experimental.pallas import tpu as pltpu
from jax.experimental.pallas import tpu_sc as plsc

K = 1024
D = 32
H = 64
COMMIT = 0.25
EPS = 1e-5

NC, NS, L = 2, 16, 16
NW = NC * NS
M = 8 * 8192
BPW = M // NW
CHUNK = 128
NCHUNK = BPW // CHUNK


def _tables_body(W1, b1, ln_g, ln_b, W2, b2, cb, qst_ref, idx_ref, err_ref):
    vals = lax.broadcasted_iota(jnp.int32, (K, 1), 0).astype(jnp.float32)
    norm = vals / (K - 1) * 2.0 - 1.0
    h = norm * W1[...] + b1[...]
    mu = jnp.mean(h, axis=1, keepdims=True)
    var = jnp.mean((h - mu) ** 2, axis=1, keepdims=True)
    h = (h - mu) / jnp.sqrt(var + EPS) * ln_g[...] + ln_b[...]
    h = jnp.maximum(h, 0.0)
    z = jnp.dot(h, W2[...], preferred_element_type=jnp.float32) + b2[...]

    c = cb[...]
    zc = lax.dot_general(z, c, (((1,), (1,)), ((), ())),
                         preferred_element_type=jnp.float32)
    dist = (jnp.sum(z * z, axis=1, keepdims=True)
            - 2.0 * zc
            + jnp.sum(c * c, axis=1)[None, :])
    col = lax.broadcasted_iota(jnp.int32, (K, K), 1)
    idx = jnp.argmin(dist, axis=1).astype(jnp.int32)
    onehot = (idx[:, None] == col).astype(jnp.float32)
    q = jnp.dot(onehot, c, preferred_element_type=jnp.float32)
    qst = z + (q - z)
    err = jnp.sum((q - z) ** 2, axis=1)

    qst_ref[...] = qst
    idx_ref[...] = idx.reshape(8, K // 8)
    err_ref[...] = err.reshape(8, K // 8)


def _build_tables(W1, b1, ln_g, ln_b, W2, b2, cb):
    qst, idx8, err8 = pl.pallas_call(
        _tables_body,
        out_shape=(
            jax.ShapeDtypeStruct((K, D), jnp.float32),
            jax.ShapeDtypeStruct((8, K // 8), jnp.int32),
            jax.ShapeDtypeStruct((8, K // 8), jnp.float32),
        ),
    )(W1, b1, ln_g, ln_b, W2, b2, cb)
    return qst, idx8.reshape(K), err8.reshape(K)


def _sc_gather(t1d, qst, idx_tab, err_tab, nb):
    mesh = plsc.VectorSubcoreMesh(core_axis_name="c", subcore_axis_name="s")
    mh = nb * 8192
    bpw = mh // NW
    nchunk = bpw // CHUNK

    @functools.partial(
        pl.kernel,
        mesh=mesh,
        compiler_params=pltpu.CompilerParams(use_tc_tiling_on_sc=False),
        out_type=(
            jax.ShapeDtypeStruct((nb, 8192, D), jnp.float32),
            jax.ShapeDtypeStruct((mh,), jnp.int32),
            jax.ShapeDtypeStruct((NW * L,), jnp.float32),
        ),
        scratch_types=[
            pltpu.VMEM((bpw,), jnp.int32),
            pltpu.VMEM_SHARED((K, D), jnp.float32),
            pltpu.VMEM_SHARED((K,), jnp.int32),
            pltpu.VMEM_SHARED((K,), jnp.float32),
            pltpu.VMEM((bpw, D), jnp.float32),
            pltpu.VMEM((bpw,), jnp.int32),
            pltpu.VMEM((bpw,), jnp.float32),
            pltpu.VMEM((L,), jnp.float32),
            pltpu.SemaphoreType.DMA,
            pltpu.SemaphoreType.DMA,
            pltpu.SemaphoreType.DMA,
        ],
    )
    def k(t_hbm, qst_hbm, idxtab_hbm, errtab_hbm,
          q_out, idx_out, part_out,
          t_v, qst_v, itab_v, etab_v, rows_v, oidx_v, oerr_v, acc_v,
          sem_a, sem_b, sem_w):
        sid = lax.axis_index("s")
        wid = sid * NC + lax.axis_index("c")
        BPW, NCHUNK = bpw, nchunk
        b = wid // (NW // nb)
        off = (wid % (NW // nb)) * BPW

        t_copy = pltpu.make_async_copy(
            t_hbm.at[pl.ds(wid * BPW, BPW)], t_v, sem_a)
        t_copy.start()

        @pl.when(sid == 0)
        def _():
            stage = [
                pltpu.make_async_copy(qst_hbm, qst_v, sem_b),
                pltpu.make_async_copy(idxtab_hbm, itab_v, sem_b),
                pltpu.make_async_copy(errtab_hbm, etab_v, sem_b),
            ]
            for c in stage:
                c.start()
            for c in stage:
                c.wait()

        t_copy.wait()
        plsc.subcore_barrier()

        small = []
        big = []
        for j in range(NCHUNK):
            idx_ref = t_v.at[pl.ds(j * CHUNK, CHUNK)]
            sl = pl.ds(j * CHUNK, CHUNK)
            small.append(pltpu.make_async_copy(
                itab_v.at[idx_ref], oidx_v.at[sl], sem_b))
            small.append(pltpu.make_async_copy(
                etab_v.at[idx_ref], oerr_v.at[sl], sem_b))
            big.append(pltpu.make_async_copy(
                qst_v.at[idx_ref], rows_v.at[sl], sem_a))
        for c in small:
            c.start()
        for c in big:
            c.start()
        for c in small:
            c.wait()

        idx_out_copy = pltpu.make_async_copy(
            oidx_v, idx_out.at[pl.ds(wid * BPW, BPW)], sem_b)
        idx_out_copy.start()

        def body(i, accs):
            a0, a1, a2, a3 = accs
            o = i * 4 * L
            return (a0 + oerr_v[pl.ds(o, L)],
                    a1 + oerr_v[pl.ds(o + L, L)],
                    a2 + oerr_v[pl.ds(o + 2 * L, L)],
                    a3 + oerr_v[pl.ds(o + 3 * L, L)])

        z16 = jnp.zeros((L,), jnp.float32)
        a0, a1, a2, a3 = lax.fori_loop(
            0, BPW // (4 * L), body, (z16, z16, z16, z16))
        acc_v[...] = (a0 + a1) + (a2 + a3)
        pltpu.sync_copy(acc_v, part_out.at[pl.ds(wid * L, L)])

        wqs = []
        for j in range(NCHUNK):
            big[j].wait()
            sl = pl.ds(j * CHUNK, CHUNK)
            wq = pltpu.make_async_copy(
                rows_v.at[sl], q_out.at[b, pl.ds(off + j * CHUNK, CHUNK)],
                sem_w)
            wq.start()
            wqs.append(wq)
        idx_out_copy.wait()
        for wq in wqs:
            wq.wait()

    return k(t1d, qst, idx_tab, err_tab)


def _loss_body(part_ref, out_ref):
    s = jnp.sum(part_ref[...], keepdims=True)
    out_ref[...] = (1.0 + COMMIT) * s.reshape(1, 1) / jnp.float32(M * D)


def _finalize_loss(partials):
    out = pl.pallas_call(
        _loss_body,
        out_shape=jax.ShapeDtypeStruct((1, 1), jnp.float32),
    )(partials)
    return out.reshape(())


def kernel(t, W1, b1, ln_g, ln_b, W2, b2, codebook):
    B, N = t.shape[0], t.shape[1]
    qst, idx_tab, err_tab = _build_tables(
        W1, b1.reshape(1, H), ln_g.reshape(1, H), ln_b.reshape(1, H),
        W2, b2.reshape(1, D), codebook)
    q3d, idx_flat, partials = _sc_gather(
        t.reshape(M), qst, idx_tab, err_tab, B)
    loss = _finalize_loss(partials.reshape(NW, L))
    return (q3d, idx_flat.reshape(B, N), loss)
